# baseline (speedup 1.0000x reference)
import functools

import jax
import jax.numpy as jnp
from jax import lax
from jax.experimental import pallas as pl
from jax.experimental.pallas import tpu as pltpu
from jax.experimental.pallas import tpu_sc as plsc

VOCAB = 100000
D_MODEL = 1024
MAX_POS = 2048
BATCH = 4
SEQ = 2048
EPS = 1e-05

NC = 2
NS = 16
NW = NC * NS
G = 16
ROWS = BATCH * SEQ
NCHUNK = 8
CROWS = ROWS // NCHUNK
RPW = CROWS // NW
NPAIR = RPW // (2 * G)
SBLK = 512


def _sc_gather_body(ids_hbm, wemb_hbm, tok_hbm,
                    idx0, idx1, rows0, rows1, gsem0, gsem1, osem0, osem1):
    wid = lax.axis_index("s") * NC + lax.axis_index("c")
    row0 = wid * RPW

    pltpu.sync_copy(ids_hbm.at[pl.ds(row0, G)], idx0)
    pltpu.async_copy(wemb_hbm.at[idx0], rows0, gsem0)

    def pipe(t, carry):
        b0 = row0 + 2 * t * G
        b1 = b0 + G
        b2 = b0 + 2 * G

        pltpu.make_async_copy(wemb_hbm.at[idx0], rows0, gsem0).wait()

        @pl.when(t > 0)
        def _():
            pltpu.make_async_copy(rows1, tok_hbm.at[pl.ds(b1 - 2 * G, G)],
                                  osem1).wait()

        pltpu.sync_copy(ids_hbm.at[pl.ds(b1, G)], idx1)
        pltpu.async_copy(wemb_hbm.at[idx1], rows1, gsem1)
        pltpu.async_copy(rows0, tok_hbm.at[pl.ds(b0, G)], osem0)

        pltpu.make_async_copy(wemb_hbm.at[idx1], rows1, gsem1).wait()

        @pl.when(t < NPAIR - 1)
        def _():
            pltpu.make_async_copy(rows0, tok_hbm.at[pl.ds(b0, G)],
                                  osem0).wait()
            pltpu.sync_copy(ids_hbm.at[pl.ds(b2, G)], idx0)
            pltpu.async_copy(wemb_hbm.at[idx0], rows0, gsem0)

        pltpu.async_copy(rows1, tok_hbm.at[pl.ds(b1, G)], osem1)
        return carry

    lax.fori_loop(0, NPAIR, pipe, 0)

    lastb = row0 + RPW - 2 * G
    pltpu.make_async_copy(rows0, tok_hbm.at[pl.ds(lastb, G)], osem0).wait()
    pltpu.make_async_copy(rows1, tok_hbm.at[pl.ds(lastb + G, G)],
                          osem1).wait()


def _sc_gather(ids_chunk, word_emb):
    mesh = plsc.VectorSubcoreMesh(core_axis_name="c", subcore_axis_name="s")
    k = functools.partial(
        pl.kernel,
        out_type=jax.ShapeDtypeStruct((CROWS, D_MODEL), jnp.float32),
        mesh=mesh,
        scratch_types=[
            pltpu.VMEM((G,), jnp.int32),
            pltpu.VMEM((G,), jnp.int32),
            pltpu.VMEM((G, D_MODEL), jnp.float32),
            pltpu.VMEM((G, D_MODEL), jnp.float32),
            pltpu.SemaphoreType.DMA,
            pltpu.SemaphoreType.DMA,
            pltpu.SemaphoreType.DMA,
            pltpu.SemaphoreType.DMA,
        ],
    )(_sc_gather_body)
    return k(ids_chunk, word_emb)


def _tc_norm_body(tok_ref, pos_ref, gamma_ref, beta_ref, out_ref):
    x = tok_ref[...] + pos_ref[...]
    mean = jnp.mean(x, axis=1, keepdims=True)
    var = jnp.mean(jnp.square(x), axis=1, keepdims=True) - mean * mean
    y = (x - mean) * lax.rsqrt(var + EPS)
    out_ref[...] = y * gamma_ref[...] + beta_ref[...]


def _tc_norm(tok_chunk, pos_emb, gamma2, beta2, s0blk):
    ns = CROWS // SBLK
    return pl.pallas_call(
        _tc_norm_body,
        grid=(ns,),
        in_specs=[
            pl.BlockSpec((SBLK, D_MODEL), lambda si: (si, 0)),
            pl.BlockSpec((SBLK, D_MODEL), lambda si, s0blk=s0blk: (s0blk + si, 0)),
            pl.BlockSpec((1, D_MODEL), lambda si: (0, 0)),
            pl.BlockSpec((1, D_MODEL), lambda si: (0, 0)),
        ],
        out_specs=pl.BlockSpec((SBLK, D_MODEL), lambda si: (si, 0)),
        out_shape=jax.ShapeDtypeStruct((CROWS, D_MODEL), jnp.float32),
    )(tok_chunk, pos_emb, gamma2, beta2)


@jax.jit
def _run(ids_flat, word_emb, pos_emb, gamma, beta):
    gamma2 = gamma.reshape(1, D_MODEL)
    beta2 = beta.reshape(1, D_MODEL)
    toks = [_sc_gather(ids_flat[c * CROWS:(c + 1) * CROWS], word_emb)
            for c in range(NCHUNK)]
    spc = SEQ // CROWS
    outs = [_tc_norm(tok, pos_emb, gamma2, beta2,
                     (c % spc) * (CROWS // SBLK))
            for c, tok in enumerate(toks)]
    return jnp.concatenate(outs, axis=0)


def kernel(input_ids, word_emb, pos_emb, gamma, beta):
    ids_flat = input_ids.reshape(-1).astype(jnp.int32)
    out = _run(ids_flat, word_emb, pos_emb, gamma, beta)
    return out.reshape(BATCH, SEQ, D_MODEL)


# speedup vs baseline: 1.1931x; 1.1931x over previous
import functools

import jax
import jax.numpy as jnp
from jax import lax
from jax.experimental import pallas as pl
from jax.experimental.pallas import tpu as pltpu
from jax.experimental.pallas import tpu_sc as plsc

VOCAB = 100000
D_MODEL = 1024
MAX_POS = 2048
BATCH = 4
SEQ = 2048
EPS = 1e-05

NC = 2
NS = 16
NW = NC * NS
G = 16
ROWS = BATCH * SEQ
CHUNKS = (2048, 6144)
SBLK = 512


def _sc_gather_body(ids_hbm, wemb_hbm, tok_hbm,
                    idx0, idx1, rows0, rows1, gsem0, gsem1, osem0, osem1,
                    rpw, npair):
    wid = lax.axis_index("s") * NC + lax.axis_index("c")
    row0 = wid * rpw

    pltpu.sync_copy(ids_hbm.at[pl.ds(row0, G)], idx0)
    pltpu.async_copy(wemb_hbm.at[idx0], rows0, gsem0)

    def pipe(t, carry):
        b0 = row0 + 2 * t * G
        b1 = b0 + G
        b2 = b0 + 2 * G

        pltpu.make_async_copy(wemb_hbm.at[idx0], rows0, gsem0).wait()

        @pl.when(t > 0)
        def _():
            pltpu.make_async_copy(rows1, tok_hbm.at[pl.ds(b1 - 2 * G, G)],
                                  osem1).wait()

        pltpu.sync_copy(ids_hbm.at[pl.ds(b1, G)], idx1)
        pltpu.async_copy(wemb_hbm.at[idx1], rows1, gsem1)
        pltpu.async_copy(rows0, tok_hbm.at[pl.ds(b0, G)], osem0)

        pltpu.make_async_copy(wemb_hbm.at[idx1], rows1, gsem1).wait()

        @pl.when(t < npair - 1)
        def _():
            pltpu.make_async_copy(rows0, tok_hbm.at[pl.ds(b0, G)],
                                  osem0).wait()
            pltpu.sync_copy(ids_hbm.at[pl.ds(b2, G)], idx0)
            pltpu.async_copy(wemb_hbm.at[idx0], rows0, gsem0)

        pltpu.async_copy(rows1, tok_hbm.at[pl.ds(b1, G)], osem1)
        return carry

    lax.fori_loop(0, npair, pipe, 0)

    lastb = row0 + rpw - 2 * G
    pltpu.make_async_copy(rows0, tok_hbm.at[pl.ds(lastb, G)], osem0).wait()
    pltpu.make_async_copy(rows1, tok_hbm.at[pl.ds(lastb + G, G)],
                          osem1).wait()


def _sc_gather(ids_chunk, word_emb, nrows):
    rpw = nrows // NW
    npair = rpw // (2 * G)
    body = functools.partial(_sc_gather_body, rpw=rpw, npair=npair)
    mesh = plsc.VectorSubcoreMesh(core_axis_name="c", subcore_axis_name="s")
    k = functools.partial(
        pl.kernel,
        out_type=jax.ShapeDtypeStruct((nrows, D_MODEL), jnp.float32),
        mesh=mesh,
        scratch_types=[
            pltpu.VMEM((G,), jnp.int32),
            pltpu.VMEM((G,), jnp.int32),
            pltpu.VMEM((G, D_MODEL), jnp.float32),
            pltpu.VMEM((G, D_MODEL), jnp.float32),
            pltpu.SemaphoreType.DMA,
            pltpu.SemaphoreType.DMA,
            pltpu.SemaphoreType.DMA,
            pltpu.SemaphoreType.DMA,
        ],
    )(body)
    return k(ids_chunk, word_emb)


def _tc_norm_body(tok_ref, pos_ref, gamma_ref, beta_ref, out_ref):
    x = tok_ref[...] + pos_ref[...]
    mean = jnp.mean(x, axis=1, keepdims=True)
    var = jnp.mean(jnp.square(x), axis=1, keepdims=True) - mean * mean
    y = (x - mean) * lax.rsqrt(var + EPS)
    out_ref[...] = y * gamma_ref[...] + beta_ref[...]


def _tc_norm(tok_chunk, pos_emb, gamma2, beta2, nb):
    ns = SEQ // SBLK
    return pl.pallas_call(
        _tc_norm_body,
        grid=(ns, nb),
        in_specs=[
            pl.BlockSpec((SBLK, D_MODEL), lambda si, bi: (bi * ns + si, 0)),
            pl.BlockSpec((SBLK, D_MODEL), lambda si, bi: (si, 0)),
            pl.BlockSpec((1, D_MODEL), lambda si, bi: (0, 0)),
            pl.BlockSpec((1, D_MODEL), lambda si, bi: (0, 0)),
        ],
        out_specs=pl.BlockSpec((SBLK, D_MODEL),
                               lambda si, bi: (bi * ns + si, 0)),
        out_shape=jax.ShapeDtypeStruct((nb * SEQ, D_MODEL), jnp.float32),
    )(tok_chunk, pos_emb, gamma2, beta2)


@jax.jit
def _run(ids_flat, word_emb, pos_emb, gamma, beta):
    gamma2 = gamma.reshape(1, D_MODEL)
    beta2 = beta.reshape(1, D_MODEL)
    toks = []
    ofs = 0
    for n in CHUNKS:
        toks.append(_sc_gather(ids_flat[ofs:ofs + n], word_emb, n))
        ofs += n
    outs = [_tc_norm(tok, pos_emb, gamma2, beta2, n // SEQ)
            for tok, n in zip(toks, CHUNKS)]
    return jnp.concatenate(outs, axis=0)


def kernel(input_ids, word_emb, pos_emb, gamma, beta):
    ids_flat = input_ids.reshape(-1).astype(jnp.int32)
    out = _run(ids_flat, word_emb, pos_emb, gamma, beta)
    return out.reshape(BATCH, SEQ, D_MODEL)


# speedup vs baseline: 1.2929x; 1.0837x over previous
import functools

import jax
import jax.numpy as jnp
from jax import lax
from jax.experimental import pallas as pl
from jax.experimental.pallas import tpu as pltpu
from jax.experimental.pallas import tpu_sc as plsc

VOCAB = 100000
D_MODEL = 1024
MAX_POS = 2048
BATCH = 4
SEQ = 2048
EPS = 1e-05

NC = 2
NS = 16
NW = NC * NS
G = 32
ROWS = BATCH * SEQ
CHUNKS = (4096, 4096)
SBLK = 1024


def _sc_gather_body(ids_hbm, wemb_hbm, tok_hbm,
                    idx0, idx1, rows0, rows1, gsem0, gsem1, osem0, osem1,
                    rpw, npair):
    wid = lax.axis_index("s") * NC + lax.axis_index("c")
    row0 = wid * rpw

    pltpu.sync_copy(ids_hbm.at[pl.ds(row0, G)], idx0)
    pltpu.async_copy(wemb_hbm.at[idx0], rows0, gsem0)

    def pipe(t, carry):
        b0 = row0 + 2 * t * G
        b1 = b0 + G
        b2 = b0 + 2 * G

        pltpu.make_async_copy(wemb_hbm.at[idx0], rows0, gsem0).wait()

        @pl.when(t > 0)
        def _():
            pltpu.make_async_copy(rows1, tok_hbm.at[pl.ds(b1 - 2 * G, G)],
                                  osem1).wait()

        pltpu.sync_copy(ids_hbm.at[pl.ds(b1, G)], idx1)
        pltpu.async_copy(wemb_hbm.at[idx1], rows1, gsem1)
        pltpu.async_copy(rows0, tok_hbm.at[pl.ds(b0, G)], osem0)

        pltpu.make_async_copy(wemb_hbm.at[idx1], rows1, gsem1).wait()

        @pl.when(t < npair - 1)
        def _():
            pltpu.make_async_copy(rows0, tok_hbm.at[pl.ds(b0, G)],
                                  osem0).wait()
            pltpu.sync_copy(ids_hbm.at[pl.ds(b2, G)], idx0)
            pltpu.async_copy(wemb_hbm.at[idx0], rows0, gsem0)

        pltpu.async_copy(rows1, tok_hbm.at[pl.ds(b1, G)], osem1)
        return carry

    lax.fori_loop(0, npair, pipe, 0)

    lastb = row0 + rpw - 2 * G
    pltpu.make_async_copy(rows0, tok_hbm.at[pl.ds(lastb, G)], osem0).wait()
    pltpu.make_async_copy(rows1, tok_hbm.at[pl.ds(lastb + G, G)],
                          osem1).wait()


def _sc_gather(ids_chunk, word_emb, nrows):
    rpw = nrows // NW
    npair = rpw // (2 * G)
    body = functools.partial(_sc_gather_body, rpw=rpw, npair=npair)
    mesh = plsc.VectorSubcoreMesh(core_axis_name="c", subcore_axis_name="s")
    k = functools.partial(
        pl.kernel,
        out_type=jax.ShapeDtypeStruct((nrows, D_MODEL), jnp.float32),
        mesh=mesh,
        scratch_types=[
            pltpu.VMEM((G,), jnp.int32),
            pltpu.VMEM((G,), jnp.int32),
            pltpu.VMEM((G, D_MODEL), jnp.float32),
            pltpu.VMEM((G, D_MODEL), jnp.float32),
            pltpu.SemaphoreType.DMA,
            pltpu.SemaphoreType.DMA,
            pltpu.SemaphoreType.DMA,
            pltpu.SemaphoreType.DMA,
        ],
    )(body)
    return k(ids_chunk, word_emb)


def _tc_norm_body(tok_ref, pos_ref, gamma_ref, beta_ref, out_ref):
    x = tok_ref[...] + pos_ref[...]
    mean = jnp.mean(x, axis=1, keepdims=True)
    var = jnp.mean(jnp.square(x), axis=1, keepdims=True) - mean * mean
    y = (x - mean) * lax.rsqrt(var + EPS)
    out_ref[...] = y * gamma_ref[...] + beta_ref[...]


def _tc_norm(tok_chunk, pos_emb, gamma2, beta2, nb):
    ns = SEQ // SBLK
    return pl.pallas_call(
        _tc_norm_body,
        grid=(ns, nb),
        in_specs=[
            pl.BlockSpec((SBLK, D_MODEL), lambda si, bi: (bi * ns + si, 0)),
            pl.BlockSpec((SBLK, D_MODEL), lambda si, bi: (si, 0)),
            pl.BlockSpec((1, D_MODEL), lambda si, bi: (0, 0)),
            pl.BlockSpec((1, D_MODEL), lambda si, bi: (0, 0)),
        ],
        out_specs=pl.BlockSpec((SBLK, D_MODEL),
                               lambda si, bi: (bi * ns + si, 0)),
        out_shape=jax.ShapeDtypeStruct((nb * SEQ, D_MODEL), jnp.float32),
    )(tok_chunk, pos_emb, gamma2, beta2)


@jax.jit
def _run(ids_flat, word_emb, pos_emb, gamma, beta):
    gamma2 = gamma.reshape(1, D_MODEL)
    beta2 = beta.reshape(1, D_MODEL)
    toks = []
    ofs = 0
    for n in CHUNKS:
        toks.append(_sc_gather(ids_flat[ofs:ofs + n], word_emb, n))
        ofs += n
    outs = [_tc_norm(tok, pos_emb, gamma2, beta2, n // SEQ)
            for tok, n in zip(toks, CHUNKS)]
    return jnp.concatenate(outs, axis=0)


def kernel(input_ids, word_emb, pos_emb, gamma, beta):
    ids_flat = input_ids.reshape(-1).astype(jnp.int32)
    out = _run(ids_flat, word_emb, pos_emb, gamma, beta)
    return out.reshape(BATCH, SEQ, D_MODEL)


# speedup vs baseline: 1.3122x; 1.0149x over previous
import functools

import jax
import jax.numpy as jnp
from jax import lax
from jax.experimental import pallas as pl
from jax.experimental.pallas import tpu as pltpu
from jax.experimental.pallas import tpu_sc as plsc

VOCAB = 100000
D_MODEL = 1024
MAX_POS = 2048
BATCH = 4
SEQ = 2048
EPS = 1e-05

NC = 2
NS = 16
NW = NC * NS
G = 32
ROWS = BATCH * SEQ
CHUNKS = (4096, 4096)
SBLK = 1024


def _sc_gather_body(ids_hbm, wemb_hbm, tok_hbm,
                    idx0, idx1, rows0, rows1, gsem0, gsem1, osem0, osem1,
                    rpw, npair):
    wid = lax.axis_index("s") * NC + lax.axis_index("c")
    row0 = wid * rpw

    pltpu.sync_copy(ids_hbm.at[pl.ds(row0, G)], idx0)
    pltpu.async_copy(wemb_hbm.at[idx0], rows0, gsem0)

    def pipe(t, carry):
        b0 = row0 + 2 * t * G
        b1 = b0 + G
        b2 = b0 + 2 * G

        pltpu.make_async_copy(wemb_hbm.at[idx0], rows0, gsem0).wait()

        @pl.when(t > 0)
        def _():
            pltpu.make_async_copy(rows1, tok_hbm.at[pl.ds(b1 - 2 * G, G)],
                                  osem1).wait()

        pltpu.sync_copy(ids_hbm.at[pl.ds(b1, G)], idx1)
        pltpu.async_copy(wemb_hbm.at[idx1], rows1, gsem1)
        pltpu.async_copy(rows0, tok_hbm.at[pl.ds(b0, G)], osem0)

        pltpu.make_async_copy(wemb_hbm.at[idx1], rows1, gsem1).wait()

        @pl.when(t < npair - 1)
        def _():
            pltpu.make_async_copy(rows0, tok_hbm.at[pl.ds(b0, G)],
                                  osem0).wait()
            pltpu.sync_copy(ids_hbm.at[pl.ds(b2, G)], idx0)
            pltpu.async_copy(wemb_hbm.at[idx0], rows0, gsem0)

        pltpu.async_copy(rows1, tok_hbm.at[pl.ds(b1, G)], osem1)
        return carry

    lax.fori_loop(0, npair, pipe, 0)

    lastb = row0 + rpw - 2 * G
    pltpu.make_async_copy(rows0, tok_hbm.at[pl.ds(lastb, G)], osem0).wait()
    pltpu.make_async_copy(rows1, tok_hbm.at[pl.ds(lastb + G, G)],
                          osem1).wait()


def _sc_gather(ids_chunk, word_emb, nrows):
    rpw = nrows // NW
    npair = rpw // (2 * G)
    body = functools.partial(_sc_gather_body, rpw=rpw, npair=npair)
    mesh = plsc.VectorSubcoreMesh(core_axis_name="c", subcore_axis_name="s")
    k = functools.partial(
        pl.kernel,
        out_type=jax.ShapeDtypeStruct((nrows, D_MODEL), jnp.float32),
        mesh=mesh,
        scratch_types=[
            pltpu.VMEM((G,), jnp.int32),
            pltpu.VMEM((G,), jnp.int32),
            pltpu.VMEM((G, D_MODEL), jnp.float32),
            pltpu.VMEM((G, D_MODEL), jnp.float32),
            pltpu.SemaphoreType.DMA,
            pltpu.SemaphoreType.DMA,
            pltpu.SemaphoreType.DMA,
            pltpu.SemaphoreType.DMA,
        ],
    )(body)
    return k(ids_chunk, word_emb)


def _tc_norm_body(tok_ref, pos_ref, gamma_ref, beta_ref, out_ref):
    x = tok_ref[...] + pos_ref[...].astype(jnp.float32)
    mean = jnp.mean(x, axis=1, keepdims=True)
    var = jnp.mean(jnp.square(x), axis=1, keepdims=True) - mean * mean
    y = (x - mean) * lax.rsqrt(var + EPS)
    out_ref[...] = y * gamma_ref[...] + beta_ref[...]


def _tc_norm(tok_chunk, pos_emb, gamma2, beta2, nb):
    ns = SEQ // SBLK
    return pl.pallas_call(
        _tc_norm_body,
        grid=(ns, nb),
        in_specs=[
            pl.BlockSpec((SBLK, D_MODEL), lambda si, bi: (bi * ns + si, 0)),
            pl.BlockSpec((SBLK, D_MODEL), lambda si, bi: (si, 0)),
            pl.BlockSpec((1, D_MODEL), lambda si, bi: (0, 0)),
            pl.BlockSpec((1, D_MODEL), lambda si, bi: (0, 0)),
        ],
        out_specs=pl.BlockSpec((SBLK, D_MODEL),
                               lambda si, bi: (bi * ns + si, 0)),
        out_shape=jax.ShapeDtypeStruct((nb * SEQ, D_MODEL), jnp.float32),
    )(tok_chunk, pos_emb, gamma2, beta2)


@jax.jit
def _run(ids_flat, word_emb, pos_emb, gamma, beta):
    pos_emb = pos_emb.astype(jnp.bfloat16)
    gamma2 = gamma.reshape(1, D_MODEL)
    beta2 = beta.reshape(1, D_MODEL)
    toks = []
    ofs = 0
    for n in CHUNKS:
        toks.append(_sc_gather(ids_flat[ofs:ofs + n], word_emb, n))
        ofs += n
    outs = [_tc_norm(tok, pos_emb, gamma2, beta2, n // SEQ)
            for tok, n in zip(toks, CHUNKS)]
    return jnp.concatenate(outs, axis=0)


def kernel(input_ids, word_emb, pos_emb, gamma, beta):
    ids_flat = input_ids.reshape(-1).astype(jnp.int32)
    out = _run(ids_flat, word_emb, pos_emb, gamma, beta)
    return out.reshape(BATCH, SEQ, D_MODEL)
